# baseline (speedup 1.0000x reference)
import dataclasses
import functools
import jax
import jax.numpy as jnp
from jax import lax
from jax.experimental import pallas as pl
from jax.experimental.pallas import tpu as pltpu
from jax.experimental.pallas import tpu_sc as plsc

N = 10000
E = 320000
D = 128
G = 8

NC = 2
NS = 16
NW = NC * NS
EPW = E // NW
CH = 80
NFULL = EPW // CH
NP = 10240
RPT = NP // NS
ZR = 128

_mesh = plsc.VectorSubcoreMesh(
    core_axis_name="c", subcore_axis_name="s", num_cores=NC, num_subcores=NS)

_no_layout_cp = pltpu.CompilerParams()
if "needs_layout_passes" in pltpu.CompilerParams.__dataclass_fields__:
    _no_layout_cp = dataclasses.replace(_no_layout_cp, needs_layout_passes=False)


def _deg_body(dst_hbm, out_hbm, idx_all, hist_v):
    cid = lax.axis_index("c")
    sid = lax.axis_index("s")
    wid = sid * NC + cid

    @pl.loop(0, NP // 16)
    def _(i):
        hist_v[pl.ds(i * 16, 16)] = jnp.zeros((16,), jnp.float32)

    pltpu.sync_copy(dst_hbm.at[pl.ds(wid * EPW, EPW)], idx_all)
    ones = jnp.ones((16,), jnp.float32)

    @pl.loop(0, EPW // 16)
    def _(i):
        v = idx_all[pl.ds(i * 16, 16)]
        plsc.addupdate_scatter(hist_v, [v], ones)

    pltpu.sync_copy(hist_v, out_hbm.at[wid])


@jax.jit
def _deg_sc(dst):
    kern = pl.kernel(
        _deg_body,
        out_type=jax.ShapeDtypeStruct((NW, NP), jnp.float32),
        mesh=_mesh,
        scratch_types=[
            pltpu.VMEM((EPW,), jnp.int32),
            pltpu.VMEM((NP,), jnp.float32),
        ],
        compiler_params=_no_layout_cp,
    )
    return kern(dst)


def _degsum_body(deg_ref, row_ref):
    row_ref[...] = jnp.sum(deg_ref[...], axis=0, keepdims=True)


@jax.jit
def _degsum_tc(deg_nw):
    return pl.pallas_call(
        _degsum_body,
        out_shape=jax.ShapeDtypeStruct((1, NP), jnp.float32),
    )(deg_nw)


def _prop_body(src_hbm, dst_hbm, tp_hbm, zeros_hbm, out_hbm,
               si0, si1, si2, si3, di0, di1, di2, di3,
               rows0, rows1, rows2, rows3, acc_sh,
               ss0, ss1, ss2, ss3, ds0, ds1, ds2, ds3, gs0, gs1, gs2, gs3):
    cid = lax.axis_index("c")
    sid = lax.axis_index("s")
    wid = sid * NC + cid
    si = (si0, si1, si2, si3)
    di = (di0, di1, di2, di3)
    rows = (rows0, rows1, rows2, rows3)
    ssem = (ss0, ss1, ss2, ss3)
    dsem = (ds0, ds1, ds2, ds3)
    gsem = (gs0, gs1, gs2, gs3)

    def start_idx(ch, b):
        base = wid * EPW + ch * CH
        pltpu.async_copy(src_hbm.at[pl.ds(base, CH)], si[b], ssem[b])
        pltpu.async_copy(dst_hbm.at[pl.ds(base, CH)], di[b], dsem[b])

    def wait_idx(b):
        pltpu.make_async_copy(src_hbm.at[pl.ds(0, CH)], si[b], ssem[b]).wait()
        pltpu.make_async_copy(dst_hbm.at[pl.ds(0, CH)], di[b], dsem[b]).wait()

    def start_gather(b):
        pltpu.async_copy(tp_hbm.at[si[b]], rows[b], gsem[b])

    def wait_gather(b):
        pltpu.make_async_copy(tp_hbm.at[si[b]], rows[b], gsem[b]).wait()

    def scatter(b):
        pltpu.sync_copy(rows[b], acc_sh.at[di[b]], add=True)

    @pl.loop(0, 5)
    def _(j):
        pltpu.sync_copy(zeros_hbm, acc_sh.at[pl.ds((sid * 5 + j) * ZR, ZR)])
    plsc.subcore_barrier()

    start_idx(0, 0)
    start_idx(1, 1)
    start_idx(2, 2)
    start_idx(3, 3)
    wait_idx(0)
    start_gather(0)
    wait_idx(1)
    start_gather(1)
    wait_idx(2)
    start_gather(2)

    @pl.loop(0, NFULL - 5, step=4)
    def _(g):
        wait_idx(3)
        start_gather(3)
        wait_gather(0)
        scatter(0)
        start_idx(g + 4, 0)

        wait_idx(0)
        start_gather(0)
        wait_gather(1)
        scatter(1)
        start_idx(g + 5, 1)

        wait_idx(1)
        start_gather(1)
        wait_gather(2)
        scatter(2)
        start_idx(g + 6, 2)

        wait_idx(2)
        start_gather(2)
        wait_gather(3)
        scatter(3)
        start_idx(g + 7, 3)

    wait_idx(3)
    start_gather(3)
    wait_gather(0)
    scatter(0)
    start_idx(NFULL - 1, 0)
    wait_gather(1)
    scatter(1)
    wait_gather(2)
    scatter(2)
    wait_idx(0)
    start_gather(0)
    wait_gather(3)
    scatter(3)
    wait_gather(0)
    scatter(0)
    plsc.subcore_barrier()

    pltpu.sync_copy(acc_sh.at[pl.ds(sid * RPT, RPT)],
                    out_hbm.at[cid, pl.ds(sid * RPT, RPT)])


@jax.jit
def _prop_sc(src, dst, tp, zeros128):
    kern = pl.kernel(
        _prop_body,
        out_type=jax.ShapeDtypeStruct((NC, NP, D), jnp.float32),
        mesh=_mesh,
        scratch_types=[
            pltpu.VMEM((CH,), jnp.int32),
            pltpu.VMEM((CH,), jnp.int32),
            pltpu.VMEM((CH,), jnp.int32),
            pltpu.VMEM((CH,), jnp.int32),
            pltpu.VMEM((CH,), jnp.int32),
            pltpu.VMEM((CH,), jnp.int32),
            pltpu.VMEM((CH,), jnp.int32),
            pltpu.VMEM((CH,), jnp.int32),
            pltpu.VMEM((CH, D), jnp.float32),
            pltpu.VMEM((CH, D), jnp.float32),
            pltpu.VMEM((CH, D), jnp.float32),
            pltpu.VMEM((CH, D), jnp.float32),
            pltpu.VMEM_SHARED((NP, D), jnp.float32),
            pltpu.SemaphoreType.DMA,
            pltpu.SemaphoreType.DMA,
            pltpu.SemaphoreType.DMA,
            pltpu.SemaphoreType.DMA,
            pltpu.SemaphoreType.DMA,
            pltpu.SemaphoreType.DMA,
            pltpu.SemaphoreType.DMA,
            pltpu.SemaphoreType.DMA,
            pltpu.SemaphoreType.DMA,
            pltpu.SemaphoreType.DMA,
            pltpu.SemaphoreType.DMA,
            pltpu.SemaphoreType.DMA,
        ],
    )
    return kern(src, dst, tp, zeros128)


def _enc_body(x_ref, we_ref, be_ref, w1_ref, t1_ref):
    a = jnp.dot(x_ref[...], we_ref[...],
                preferred_element_type=jnp.float32) + be_ref[...]
    h = jnp.where(a > 0, a, jnp.exp(jnp.minimum(a, 0.0)) - 1.0)
    t1_ref[...] = jnp.dot(h, w1_ref[...], preferred_element_type=jnp.float32)


@jax.jit
def _enc_tc(x, W_enc, b_enc, W1):
    return pl.pallas_call(
        _enc_body,
        out_shape=jax.ShapeDtypeStruct((N, D), jnp.float32),
    )(x, W_enc, b_enc, W1)


def _dinv_body(degcol_ref, t1_ref, dinv_ref, tp1_ref):
    dinv = lax.rsqrt(degcol_ref[...] + 1.0)
    dinv_ref[...] = dinv
    tp1_ref[...] = dinv * t1_ref[...]


@jax.jit
def _dinv_tc(degcol, t1):
    return pl.pallas_call(
        _dinv_body,
        out_shape=(jax.ShapeDtypeStruct((N, 1), jnp.float32),
                   jax.ShapeDtypeStruct((N, D), jnp.float32)),
    )(degcol, t1)


def _comb_body(s_ref, t_ref, b_ref, dinv_ref, wn_ref, tn_ref, tpn_ref):
    dinv = dinv_ref[...]
    u = (dinv * (s_ref[0, :N] + s_ref[1, :N])
         + dinv * dinv * t_ref[...] + b_ref[...])
    u = jnp.maximum(u, 0.0)
    tn = jnp.dot(u, wn_ref[...], preferred_element_type=jnp.float32)
    tn_ref[...] = tn
    tpn_ref[...] = dinv * tn


@jax.jit
def _comb_tc(s, t, b, dinv, Wn):
    return pl.pallas_call(
        _comb_body,
        out_shape=(jax.ShapeDtypeStruct((N, D), jnp.float32),
                   jax.ShapeDtypeStruct((N, D), jnp.float32)),
    )(s, t, b, dinv, Wn)


def _final_body(s_ref, t_ref, b_ref, dinv_ref, batch_ref,
                wfc_ref, bfc_ref, wpr_ref, bpr_ref, out_ref):
    dinv = dinv_ref[...]
    h = (dinv * (s_ref[0, :N] + s_ref[1, :N])
         + dinv * dinv * t_ref[...] + b_ref[...])
    h = jnp.maximum(h, 0.0)
    batch = batch_ref[...]
    neg = jnp.float32(-jnp.inf)
    rows = []
    for g in range(G):
        m = jnp.where(batch == g, h, neg)
        rows.append(jnp.max(m, axis=0, keepdims=True))
    pooled = jnp.concatenate(rows, axis=0)
    pooled = jnp.where(jnp.isfinite(pooled), pooled, 0.0)
    z = jnp.tanh(jnp.dot(pooled, wfc_ref[...],
                         preferred_element_type=jnp.float32) + bfc_ref[...])
    out_ref[...] = jax.nn.sigmoid(
        jnp.dot(z, wpr_ref[...], preferred_element_type=jnp.float32)
        + bpr_ref[...])


@jax.jit
def _final_tc(s, t, b, dinv, batch2d, Wfc_p, bfc_p, Wpr_p, bpr_p):
    return pl.pallas_call(
        _final_body,
        out_shape=jax.ShapeDtypeStruct((G, 128), jnp.float32),
    )(s, t, b, dinv, batch2d, Wfc_p, bfc_p, Wpr_p, bpr_p)


@jax.jit
def kernel(x, edge_index, batch, W_enc, b_enc, W1, b1, W2, b2, W3, b3,
           W_fc, b_fc, W_pred, b_pred):
    src = edge_index[0]
    dst = edge_index[1]
    zeros128 = jnp.zeros((ZR, D), jnp.float32)
    batch2d = batch.reshape(N, 1)

    Wfc_p = jnp.zeros((D, 128), jnp.float32).at[:, :64].set(W_fc)
    bfc_p = jnp.zeros((1, 128), jnp.float32).at[0, :64].set(b_fc)
    Wpr_p = jnp.zeros((128, 128), jnp.float32).at[:64, :2].set(W_pred)
    bpr_p = jnp.zeros((1, 128), jnp.float32).at[0, :2].set(b_pred)

    deg_nw = _deg_sc(dst)
    t1 = _enc_tc(x, W_enc, b_enc.reshape(1, D), W1)
    deg_row = _degsum_tc(deg_nw)
    degcol = deg_row.reshape(NP, 1)[:N]
    dinv, tp1 = _dinv_tc(degcol, t1)

    s1 = _prop_sc(src, dst, tp1, zeros128)
    t2, tp2 = _comb_tc(s1, t1, b1.reshape(1, D), dinv, W2)
    s2 = _prop_sc(src, dst, tp2, zeros128)
    t3, tp3 = _comb_tc(s2, t2, b2.reshape(1, D), dinv, W3)
    s3 = _prop_sc(src, dst, tp3, zeros128)

    out_pad = _final_tc(s3, t3, b3.reshape(1, D), dinv, batch2d,
                        Wfc_p, bfc_p, Wpr_p, bpr_p)
    return out_pad[:, :2]


# speedup vs baseline: 1.0400x; 1.0400x over previous
import dataclasses
import functools
import jax
import jax.numpy as jnp
from jax import lax
from jax.experimental import pallas as pl
from jax.experimental.pallas import tpu as pltpu
from jax.experimental.pallas import tpu_sc as plsc

N = 10000
E = 320000
D = 128
G = 8

NC = 2
NS = 16
NW = NC * NS
EPW = E // NW
CH = 80
NFULL = EPW // CH
NP = 10240
RPT = NP // NS
ZR = 128

_mesh = plsc.VectorSubcoreMesh(
    core_axis_name="c", subcore_axis_name="s", num_cores=NC, num_subcores=NS)

_no_layout_cp = pltpu.CompilerParams()
if "needs_layout_passes" in pltpu.CompilerParams.__dataclass_fields__:
    _no_layout_cp = dataclasses.replace(_no_layout_cp, needs_layout_passes=False)


def _deg_body(dst_hbm, out_hbm, idx_all, hist_v):
    cid = lax.axis_index("c")
    sid = lax.axis_index("s")
    wid = sid * NC + cid

    @pl.loop(0, NP // 16)
    def _(i):
        hist_v[pl.ds(i * 16, 16)] = jnp.zeros((16,), jnp.float32)

    pltpu.sync_copy(dst_hbm.at[pl.ds(wid * EPW, EPW)], idx_all)
    ones = jnp.ones((16,), jnp.float32)

    @pl.loop(0, EPW // 16)
    def _(i):
        v = idx_all[pl.ds(i * 16, 16)]
        plsc.addupdate_scatter(hist_v, [v], ones)

    pltpu.sync_copy(hist_v, out_hbm.at[wid])


@jax.jit
def _deg_sc(dst):
    kern = pl.kernel(
        _deg_body,
        out_type=jax.ShapeDtypeStruct((NW, NP), jnp.float32),
        mesh=_mesh,
        scratch_types=[
            pltpu.VMEM((EPW,), jnp.int32),
            pltpu.VMEM((NP,), jnp.float32),
        ],
        compiler_params=_no_layout_cp,
    )
    return kern(dst)


def _degsum_body(deg_ref, row_ref):
    row_ref[...] = jnp.sum(deg_ref[...], axis=0, keepdims=True)


@jax.jit
def _degsum_tc(deg_nw):
    return pl.pallas_call(
        _degsum_body,
        out_shape=jax.ShapeDtypeStruct((1, NP), jnp.float32),
    )(deg_nw)


def _prop_body(src_hbm, dst_hbm, tp_hbm, zeros_hbm, out_hbm,
               si0, si1, si2, di0, di1, di2, rows0, rows1, rows2,
               zero_v, acc_sh,
               ss0, ss1, ss2, ds0, ds1, ds2, gs0, gs1, gs2):
    cid = lax.axis_index("c")
    sid = lax.axis_index("s")
    wid = sid * NC + cid
    si = (si0, si1, si2)
    di = (di0, di1, di2)
    rows = (rows0, rows1, rows2)
    ssem = (ss0, ss1, ss2)
    dsem = (ds0, ds1, ds2)
    gsem = (gs0, gs1, gs2)

    def start_idx(ch, b):
        base = wid * EPW + ch * CH
        pltpu.async_copy(src_hbm.at[pl.ds(base, CH)], si[b], ssem[b])
        pltpu.async_copy(dst_hbm.at[pl.ds(base, CH)], di[b], dsem[b])

    def wait_idx(b):
        pltpu.make_async_copy(src_hbm.at[pl.ds(0, CH)], si[b], ssem[b]).wait()
        pltpu.make_async_copy(dst_hbm.at[pl.ds(0, CH)], di[b], dsem[b]).wait()

    def start_gather(b):
        pltpu.async_copy(tp_hbm.at[si[b]], rows[b], gsem[b])

    def wait_gather(b):
        pltpu.make_async_copy(tp_hbm.at[si[b]], rows[b], gsem[b]).wait()

    pltpu.sync_copy(zeros_hbm, zero_v)

    @pl.loop(0, 5)
    def _(j):
        pltpu.sync_copy(zero_v, acc_sh.at[pl.ds((sid * 5 + j) * ZR, ZR)])
    plsc.subcore_barrier()

    start_idx(0, 0)
    start_idx(1, 1)
    start_idx(2, 2)
    wait_idx(0)
    start_gather(0)
    wait_idx(1)
    start_gather(1)

    @pl.loop(0, NFULL - 2, step=3)
    def _(g):
        wait_idx(2)
        start_gather(2)
        wait_gather(0)
        pltpu.sync_copy(rows[0], acc_sh.at[di[0]], add=True)
        start_idx(g + 3, 0)

        wait_idx(0)
        start_gather(0)
        wait_gather(1)
        pltpu.sync_copy(rows[1], acc_sh.at[di[1]], add=True)
        start_idx(g + 4, 1)

        wait_idx(1)
        start_gather(1)
        wait_gather(2)
        pltpu.sync_copy(rows[2], acc_sh.at[di[2]], add=True)

        @pl.when(g < NFULL - 5)
        def _():
            start_idx(g + 5, 2)

    wait_gather(0)
    pltpu.sync_copy(rows[0], acc_sh.at[di[0]], add=True)
    wait_gather(1)
    pltpu.sync_copy(rows[1], acc_sh.at[di[1]], add=True)
    plsc.subcore_barrier()

    pltpu.sync_copy(acc_sh.at[pl.ds(sid * RPT, RPT)],
                    out_hbm.at[cid, pl.ds(sid * RPT, RPT)])


@jax.jit
def _prop_sc(src, dst, tp, zeros128):
    kern = pl.kernel(
        _prop_body,
        out_type=jax.ShapeDtypeStruct((NC, NP, D), jnp.float32),
        mesh=_mesh,
        scratch_types=[
            pltpu.VMEM((CH,), jnp.int32),
            pltpu.VMEM((CH,), jnp.int32),
            pltpu.VMEM((CH,), jnp.int32),
            pltpu.VMEM((CH,), jnp.int32),
            pltpu.VMEM((CH,), jnp.int32),
            pltpu.VMEM((CH,), jnp.int32),
            pltpu.VMEM((CH, D), jnp.float32),
            pltpu.VMEM((CH, D), jnp.float32),
            pltpu.VMEM((CH, D), jnp.float32),
            pltpu.VMEM((ZR, D), jnp.float32),
            pltpu.VMEM_SHARED((NP, D), jnp.float32),
            pltpu.SemaphoreType.DMA,
            pltpu.SemaphoreType.DMA,
            pltpu.SemaphoreType.DMA,
            pltpu.SemaphoreType.DMA,
            pltpu.SemaphoreType.DMA,
            pltpu.SemaphoreType.DMA,
            pltpu.SemaphoreType.DMA,
            pltpu.SemaphoreType.DMA,
            pltpu.SemaphoreType.DMA,
        ],
    )
    return kern(src, dst, tp, zeros128)


def _enc_body(x_ref, we_ref, be_ref, w1_ref, t1_ref):
    a = jnp.dot(x_ref[...], we_ref[...],
                preferred_element_type=jnp.float32) + be_ref[...]
    h = jnp.where(a > 0, a, jnp.exp(jnp.minimum(a, 0.0)) - 1.0)
    t1_ref[...] = jnp.dot(h, w1_ref[...], preferred_element_type=jnp.float32)


@jax.jit
def _enc_tc(x, W_enc, b_enc, W1):
    return pl.pallas_call(
        _enc_body,
        out_shape=jax.ShapeDtypeStruct((N, D), jnp.float32),
    )(x, W_enc, b_enc, W1)


def _dinv_body(degcol_ref, t1_ref, dinv_ref, tp1_ref):
    dinv = lax.rsqrt(degcol_ref[...] + 1.0)
    dinv_ref[...] = dinv
    tp1_ref[...] = dinv * t1_ref[...]


@jax.jit
def _dinv_tc(degcol, t1):
    return pl.pallas_call(
        _dinv_body,
        out_shape=(jax.ShapeDtypeStruct((N, 1), jnp.float32),
                   jax.ShapeDtypeStruct((N, D), jnp.float32)),
    )(degcol, t1)


def _comb_body(s_ref, t_ref, b_ref, dinv_ref, wn_ref, tn_ref, tpn_ref):
    dinv = dinv_ref[...]
    u = (dinv * (s_ref[0, :N] + s_ref[1, :N])
         + dinv * dinv * t_ref[...] + b_ref[...])
    u = jnp.maximum(u, 0.0)
    tn = jnp.dot(u, wn_ref[...], preferred_element_type=jnp.float32)
    tn_ref[...] = tn
    tpn_ref[...] = dinv * tn


@jax.jit
def _comb_tc(s, t, b, dinv, Wn):
    return pl.pallas_call(
        _comb_body,
        out_shape=(jax.ShapeDtypeStruct((N, D), jnp.float32),
                   jax.ShapeDtypeStruct((N, D), jnp.float32)),
    )(s, t, b, dinv, Wn)


def _final_body(s_ref, t_ref, b_ref, dinv_ref, batch_ref,
                wfc_ref, bfc_ref, wpr_ref, bpr_ref, out_ref):
    dinv = dinv_ref[...]
    h = (dinv * (s_ref[0, :N] + s_ref[1, :N])
         + dinv * dinv * t_ref[...] + b_ref[...])
    h = jnp.maximum(h, 0.0)
    batch = batch_ref[...]
    neg = jnp.float32(-jnp.inf)
    rows = []
    for g in range(G):
        m = jnp.where(batch == g, h, neg)
        rows.append(jnp.max(m, axis=0, keepdims=True))
    pooled = jnp.concatenate(rows, axis=0)
    pooled = jnp.where(jnp.isfinite(pooled), pooled, 0.0)
    z = jnp.tanh(jnp.dot(pooled, wfc_ref[...],
                         preferred_element_type=jnp.float32) + bfc_ref[...])
    out_ref[...] = jax.nn.sigmoid(
        jnp.dot(z, wpr_ref[...], preferred_element_type=jnp.float32)
        + bpr_ref[...])


@jax.jit
def _final_tc(s, t, b, dinv, batch2d, Wfc_p, bfc_p, Wpr_p, bpr_p):
    return pl.pallas_call(
        _final_body,
        out_shape=jax.ShapeDtypeStruct((G, 128), jnp.float32),
    )(s, t, b, dinv, batch2d, Wfc_p, bfc_p, Wpr_p, bpr_p)


@jax.jit
def kernel(x, edge_index, batch, W_enc, b_enc, W1, b1, W2, b2, W3, b3,
           W_fc, b_fc, W_pred, b_pred):
    src = edge_index[0]
    dst = edge_index[1]
    zeros128 = jnp.zeros((ZR, D), jnp.float32)
    batch2d = batch.reshape(N, 1)

    Wfc_p = jnp.zeros((D, 128), jnp.float32).at[:, :64].set(W_fc)
    bfc_p = jnp.zeros((1, 128), jnp.float32).at[0, :64].set(b_fc)
    Wpr_p = jnp.zeros((128, 128), jnp.float32).at[:64, :2].set(W_pred)
    bpr_p = jnp.zeros((1, 128), jnp.float32).at[0, :2].set(b_pred)

    deg_nw = _deg_sc(dst)
    t1 = _enc_tc(x, W_enc, b_enc.reshape(1, D), W1)
    deg_row = _degsum_tc(deg_nw)
    degcol = deg_row.reshape(NP, 1)[:N]
    dinv, tp1 = _dinv_tc(degcol, t1)

    s1 = _prop_sc(src, dst, tp1, zeros128)
    t2, tp2 = _comb_tc(s1, t1, b1.reshape(1, D), dinv, W2)
    s2 = _prop_sc(src, dst, tp2, zeros128)
    t3, tp3 = _comb_tc(s2, t2, b2.reshape(1, D), dinv, W3)
    s3 = _prop_sc(src, dst, tp3, zeros128)

    out_pad = _final_tc(s3, t3, b3.reshape(1, D), dinv, batch2d,
                        Wfc_p, bfc_p, Wpr_p, bpr_p)
    return out_pad[:, :2]
